# baseline (speedup 1.0000x reference)
import functools
import numpy as np
import jax
import jax.numpy as jnp
from jax import lax
from jax.experimental import pallas as pl
from jax.experimental.pallas import tpu as pltpu
from jax.experimental.pallas import tpu_sc as plsc

_A = np.float32(1.0 / np.sqrt(8.0))
_MAGIC = np.float32(1.5 * 2.0**23)
_Y = np.float32(4.0) * _A
_NTOK = 512
_K = 8192
_D = 24
_NCORE = 2
_NSUB = 16
_NW = _NCORE * _NSUB
_TPW = _NTOK // _NW

_SSC = 3584
_NBLK = _SSC // 16
_NTC = _K - _SSC
_KB = 512
_NKB = _NTC // _KB

_SUPPORTS = (
    (0, 1, 2, 3, 4, 5, 6, 7),
    (0, 1, 2, 3, 8, 9, 10, 11),
    (0, 1, 4, 5, 8, 9, 12, 13),
    (0, 2, 4, 6, 8, 10, 12, 14),
)
_INNER_TZ = tuple((i + 1 & -(i + 1)).bit_length() - 1 for i in range(15))

_G12 = np.array([
    [1,1,1,1,1,1,1,1,0,0,0,0,0,0,0,0,0,0,0,0,0,0,0,0],
    [1,1,1,1,0,0,0,0,1,1,1,1,0,0,0,0,0,0,0,0,0,0,0,0],
    [1,1,0,0,1,1,0,0,1,1,0,0,1,1,0,0,0,0,0,0,0,0,0,0],
    [1,0,1,0,1,0,1,0,1,0,1,0,1,0,1,0,0,0,0,0,0,0,0,0],
    [1,0,0,1,1,0,0,1,1,0,0,1,1,0,0,1,0,0,0,0,0,0,0,0],
    [1,0,1,0,1,0,0,1,1,1,0,0,0,0,0,0,1,1,0,0,0,0,0,0],
    [1,0,0,1,1,1,0,0,1,0,1,0,0,0,0,0,1,0,1,0,0,0,0,0],
    [1,1,0,0,1,0,1,0,1,0,0,1,0,0,0,0,1,0,0,1,0,0,0,0],
    [0,1,1,1,1,0,0,0,1,0,0,0,1,0,0,0,1,0,0,0,1,0,0,0],
    [0,0,0,0,0,0,0,0,1,1,0,0,1,1,0,0,1,1,0,0,1,1,0,0],
    [0,0,0,0,0,0,0,0,1,0,1,0,1,0,1,0,1,0,1,0,1,0,1,0],
    [1,1,1,1,1,1,1,1,1,1,1,1,1,1,1,1,1,1,1,1,1,1,1,1]], dtype=np.int64)
_G12H = (_G12 * 0.5).astype(np.float32)
_U4 = (np.array([-3] + [1] * 23) * 0.25).astype(np.float32)


def _gray_to_k(g):
    k = g & 4096
    for b in range(12):
        k = k | (((g >> b) & 1) << (11 - b))
    return k


def _np_step_to_k(s):
    g = s ^ (s >> 1)
    return _gray_to_k(g)


_PERM = np.array([_np_step_to_k(s) for s in range(_K)], dtype=np.int32)
_KEND = _PERM[np.arange(_NBLK - 1) * 16 + 15]
_KNXT = _PERM[np.arange(_NBLK - 1) * 16 + 16]


def _round_ne(t):
    return (t + _MAGIC) - _MAGIC


def _lanes(x):
    return jnp.broadcast_to(x, (_NSUB,))


def _sc_body(xT_hbm, cflat_hbm, delta_hbm, outT_hbm, bd_hbm, bk_hbm,
             xT_v, delta_v, outT_v, crowT_v, bd_v, bk_v, sem):
    cid = lax.axis_index("c")
    sid = lax.axis_index("s")
    wid = sid * _NCORE + cid
    base = wid * _TPW

    pltpu.sync_copy(xT_hbm.at[:, pl.ds(base, _TPW)], xT_v)
    pltpu.sync_copy(delta_hbm, delta_v)

    qs = [jnp.zeros((_TPW,), jnp.float32) for _ in range(_D)]
    sb = [jnp.full((_TPW,), -0.5, jnp.float32) for _ in range(4)]
    bd = jnp.full((_TPW,), jnp.inf, jnp.float32)
    bs = jnp.zeros((_TPW,), jnp.int32)

    def mbody(m, carry):
        qs, sb, bd, bs = [list(carry[0]), list(carry[1])] + list(carry[2:])
        s16 = m * 16
        for i in range(16):
            a = [jnp.zeros((_TPW,), jnp.float32) for _ in range(3)]
            p = [jnp.zeros((_TPW,), jnp.float32) for _ in range(3)]
            mm = [jnp.zeros((_TPW,), jnp.float32) for _ in range(3)]
            for j in range(_D):
                t = xT_v[j, :] + qs[j]
                r = _round_ne(t)
                e = t - r
                u = j % 3
                a[u] = a[u] + e * e
                p[u] = p[u] + r
                mm[u] = jnp.maximum(mm[u], jnp.abs(e))
            acc = (a[0] + a[1]) + a[2]
            ps = (p[0] + p[1]) + p[2]
            em = jnp.maximum(jnp.maximum(mm[0], mm[1]), mm[2])
            h = ps * 0.5
            odd = jnp.abs(h - _round_ne(h)) * 2.0
            score = acc + (1.0 - 2.0 * em) * odd
            better = score < bd
            bd = jnp.where(better, score, bd)
            bs = jnp.where(better, _lanes(s16 + i), bs)
            if i < 15:
                b = _INNER_TZ[i]
                for j in _SUPPORTS[b]:
                    qs[j] = qs[j] + sb[b]
                sb[b] = -sb[b]
            else:
                d0 = delta_v[pl.ds(m * _D, _NSUB)]
                d1 = delta_v[pl.ds(m * _D + 8, _NSUB)]
                for j in range(_D):
                    dj = _lanes(d0[j] if j < _NSUB else d1[j - 8])
                    qs[j] = qs[j] + dj
        return (tuple(qs), tuple(sb), bd, bs)

    _, _, bd, bs = lax.fori_loop(
        0, _NBLK, mbody, (tuple(qs), tuple(sb), bd, bs))

    g = bs ^ (bs >> 1)
    bk = g & 4096
    for b in range(12):
        bk = bk | (((g >> b) & 1) << (11 - b))

    bd_v[...] = bd
    bk_v[...] = bk
    pltpu.sync_copy(bd_v, bd_hbm.at[pl.ds(base, _TPW)])
    pltpu.sync_copy(bk_v, bk_hbm.at[pl.ds(base, _TPW)])

    descs = []
    for j in range(_D):
        idx = bk * _D + j
        descs.append(pltpu.async_copy(cflat_hbm.at[idx], crowT_v.at[j], sem))
    for d in descs:
        d.wait()

    ps = jnp.zeros((_TPW,), jnp.float32)
    mx = jnp.full((_TPW,), -1.0, jnp.float32)
    col = jnp.zeros((_TPW,), jnp.int32)
    for j in range(_D):
        cj = crowT_v[j, :]
        t = xT_v[j, :] - cj
        r = _round_ne(t)
        ea = jnp.abs(t - r)
        ps = ps + r
        upd = ea > mx
        mx = jnp.where(upd, ea, mx)
        col = jnp.where(upd, _lanes(jnp.int32(j)), col)
    h = ps * 0.5
    odd = jnp.abs(h - _round_ne(h)) * 2.0 > 0.5
    for j in range(_D):
        cj = crowT_v[j, :]
        t = xT_v[j, :] - cj
        r = _round_ne(t)
        e = t - r
        f = jnp.where((col == j) & odd, r + jnp.sign(e), r)
        outT_v[j, :] = (f + cj) * _Y

    pltpu.sync_copy(outT_v, outT_hbm.at[:, pl.ds(base, _TPW)])


@functools.cache
def _make_sc_call(interpret=False):
    mesh = plsc.VectorSubcoreMesh(core_axis_name="c", subcore_axis_name="s",
                                  num_cores=_NCORE, num_subcores=_NSUB)
    return pl.kernel(
        _sc_body,
        out_type=(
            jax.ShapeDtypeStruct((_D, _NTOK), jnp.float32),
            jax.ShapeDtypeStruct((_NTOK,), jnp.float32),
            jax.ShapeDtypeStruct((_NTOK,), jnp.int32),
        ),
        mesh=mesh,
        scratch_types=[
            pltpu.VMEM((_D, _TPW), jnp.float32),
            pltpu.VMEM((_NBLK * _D,), jnp.float32),
            pltpu.VMEM((_D, _TPW), jnp.float32),
            pltpu.VMEM((_D, _TPW), jnp.float32),
            pltpu.VMEM((_TPW,), jnp.float32),
            pltpu.VMEM((_TPW,), jnp.int32),
            pltpu.SemaphoreType.DMA,
        ],
        compiler_params=pltpu.CompilerParams(use_tc_tiling_on_sc=False),
        interpret=interpret,
    )


def _tc_body(x4_ref, c4t_ref, kmap_ref, g12h_ref, u4_ref,
             y_ref, bd_ref, bk_ref, bda_ref, bka_ref):
    def tb_body(tb, _):
        xb = x4_ref[pl.ds(tb * 8, 8), :]
        xjs = [xb[:, j:j + 1] for j in range(_D)]

        def kb_body(kb, carry):
            bd, bk = carry
            a = jnp.zeros((8, _KB), jnp.float32)
            p = jnp.zeros((8, _KB), jnp.float32)
            mm = jnp.zeros((8, _KB), jnp.float32)
            for j in range(_D):
                cj = c4t_ref[pl.ds(j, 1), pl.ds(kb * _KB, _KB)]
                t = xjs[j] - cj
                r = _round_ne(t)
                e = t - r
                a = a + e * e
                p = p + r
                mm = jnp.maximum(mm, jnp.abs(e))
            h = p * 0.5
            odd = jnp.abs(h - _round_ne(h)) * 2.0
            score = a + (1.0 - 2.0 * mm) * odd
            kv = jnp.broadcast_to(
                kmap_ref[pl.ds(0, 1), pl.ds(kb * _KB, _KB)], (8, _KB))
            better = score < bd
            return (jnp.where(better, score, bd),
                    jnp.where(better, kv, bk))

        bd, bk = lax.fori_loop(
            0, _NKB, kb_body,
            (jnp.full((8, _KB), jnp.inf, jnp.float32),
             jnp.zeros((8, _KB), jnp.int32)))
        bda_ref[pl.ds(tb * 8, 8), :] = bd
        bka_ref[pl.ds(tb * 8, 8), :] = bk
        return 0

    lax.fori_loop(0, _NTOK // 8, tb_body, 0)

    bd = bda_ref[...]
    bk = bka_ref[...]
    m = jnp.min(bd, axis=1, keepdims=True)
    kw = jnp.min(jnp.where(bd == m, bk, jnp.int32(1 << 30)),
                 axis=1, keepdims=True)

    c4w = jnp.where(kw >= 4096, u4_ref[pl.ds(0, 1), :], 0.0)
    for b in range(12):
        bit = ((kw >> (11 - b)) & 1).astype(jnp.float32)
        c4w = c4w + bit * g12h_ref[pl.ds(b, 1), :]
    i24 = lax.broadcasted_iota(jnp.int32, (_NTOK, _D), 1)
    xall = x4_ref[...]
    t = xall - c4w
    r = _round_ne(t)
    e = t - r
    ea = jnp.abs(e)
    psum = jnp.sum(r, axis=1, keepdims=True)
    hh = psum * 0.5
    odd = jnp.abs(hh - _round_ne(hh)) * 2.0 > 0.5
    mx = jnp.max(ea, axis=1, keepdims=True)
    colv = jnp.min(jnp.where(ea == mx, i24, 999), axis=1, keepdims=True)
    f = jnp.where((i24 == colv) & odd, r + jnp.sign(e), r)
    y_ref[...] = (f + c4w) * _Y
    bd_ref[...] = m
    bk_ref[...] = kw


@functools.cache
def _make_tc_call():
    return pl.pallas_call(
        _tc_body,
        out_shape=(
            jax.ShapeDtypeStruct((_NTOK, _D), jnp.float32),
            jax.ShapeDtypeStruct((_NTOK, 1), jnp.float32),
            jax.ShapeDtypeStruct((_NTOK, 1), jnp.int32),
        ),
        scratch_shapes=[
            pltpu.VMEM((_NTOK, _KB), jnp.float32),
            pltpu.VMEM((_NTOK, _KB), jnp.int32),
        ],
    )


@jax.jit
def kernel(x_in, C_rep):
    x = x_in / _A
    x4 = x * 0.25
    xT4 = x4.T
    c4 = C_rep.astype(jnp.float32) * 0.25
    cflat = c4.reshape(-1)
    delta = jnp.concatenate(
        [c4[_KEND] - c4[_KNXT], jnp.zeros((1, _D), jnp.float32)], axis=0)
    c4t_tc = c4.T[:, _PERM[_SSC:]]
    kmap = jnp.asarray(_PERM[_SSC:][None, :])

    yT_sc, bd_sc, bk_sc = _make_sc_call()(xT4, cflat, delta.reshape(-1))
    y_tc, bd_tc, bk_tc = _make_tc_call()(
        x4, c4t_tc, kmap, jnp.asarray(_G12H), jnp.asarray(_U4)[None, :])

    bd_tc = bd_tc[:, 0]
    bk_tc = bk_tc[:, 0]
    use_tc = (bd_tc < bd_sc) | ((bd_tc == bd_sc) & (bk_tc < bk_sc))
    return jnp.where(use_tc[:, None], y_tc, yT_sc.T)


# speedup vs baseline: 1.0009x; 1.0009x over previous
import functools
import numpy as np
import jax
import jax.numpy as jnp
from jax import lax
from jax.experimental import pallas as pl
from jax.experimental.pallas import tpu as pltpu
from jax.experimental.pallas import tpu_sc as plsc

_A = np.float32(1.0 / np.sqrt(8.0))
_MAGIC = np.float32(1.5 * 2.0**23)
_Y = np.float32(4.0) * _A
_NTOK = 512
_K = 8192
_D = 24
_NCORE = 2
_NSUB = 16
_NW = _NCORE * _NSUB
_TPW = _NTOK // _NW

_SSC = 3584
_NBLK = _SSC // 16
_NTC = _K - _SSC
_KB = 512
_NKB = _NTC // _KB

_SUPPORTS = (
    (0, 1, 2, 3, 4, 5, 6, 7),
    (0, 1, 2, 3, 8, 9, 10, 11),
    (0, 1, 4, 5, 8, 9, 12, 13),
    (0, 2, 4, 6, 8, 10, 12, 14),
)
_INNER_TZ = tuple((i + 1 & -(i + 1)).bit_length() - 1 for i in range(15))

_G12 = np.array([
    [1,1,1,1,1,1,1,1,0,0,0,0,0,0,0,0,0,0,0,0,0,0,0,0],
    [1,1,1,1,0,0,0,0,1,1,1,1,0,0,0,0,0,0,0,0,0,0,0,0],
    [1,1,0,0,1,1,0,0,1,1,0,0,1,1,0,0,0,0,0,0,0,0,0,0],
    [1,0,1,0,1,0,1,0,1,0,1,0,1,0,1,0,0,0,0,0,0,0,0,0],
    [1,0,0,1,1,0,0,1,1,0,0,1,1,0,0,1,0,0,0,0,0,0,0,0],
    [1,0,1,0,1,0,0,1,1,1,0,0,0,0,0,0,1,1,0,0,0,0,0,0],
    [1,0,0,1,1,1,0,0,1,0,1,0,0,0,0,0,1,0,1,0,0,0,0,0],
    [1,1,0,0,1,0,1,0,1,0,0,1,0,0,0,0,1,0,0,1,0,0,0,0],
    [0,1,1,1,1,0,0,0,1,0,0,0,1,0,0,0,1,0,0,0,1,0,0,0],
    [0,0,0,0,0,0,0,0,1,1,0,0,1,1,0,0,1,1,0,0,1,1,0,0],
    [0,0,0,0,0,0,0,0,1,0,1,0,1,0,1,0,1,0,1,0,1,0,1,0],
    [1,1,1,1,1,1,1,1,1,1,1,1,1,1,1,1,1,1,1,1,1,1,1,1]], dtype=np.int64)
_G12H = (_G12 * 0.5).astype(np.float32)
_U4 = (np.array([-3] + [1] * 23) * 0.25).astype(np.float32)


def _gray_to_k(g):
    k = g & 4096
    for b in range(12):
        k = k | (((g >> b) & 1) << (11 - b))
    return k


def _np_step_to_k(s):
    g = s ^ (s >> 1)
    return _gray_to_k(g)


_PERM = np.array([_np_step_to_k(s) for s in range(_K)], dtype=np.int32)
_KEND = _PERM[np.arange(_NBLK - 1) * 16 + 15]
_KNXT = _PERM[np.arange(_NBLK - 1) * 16 + 16]


def _round_ne(t):
    return (t + _MAGIC) - _MAGIC


def _lanes(x):
    return jnp.broadcast_to(x, (_NSUB,))


def _sc_body(xT_hbm, cflat_hbm, delta_hbm, outT_hbm, bd_hbm, bk_hbm,
             xT_v, delta_v, outT_v, crowT_v, bd_v, bk_v, sem):
    cid = lax.axis_index("c")
    sid = lax.axis_index("s")
    wid = sid * _NCORE + cid
    base = wid * _TPW

    pltpu.sync_copy(xT_hbm.at[:, pl.ds(base, _TPW)], xT_v)
    pltpu.sync_copy(delta_hbm, delta_v)

    qs = [jnp.zeros((_TPW,), jnp.float32) for _ in range(_D)]
    sb = [jnp.full((_TPW,), -0.5, jnp.float32) for _ in range(4)]
    bd = jnp.full((_TPW,), jnp.inf, jnp.float32)
    bs = jnp.zeros((_TPW,), jnp.int32)

    def mbody(m, carry):
        qs, sb, bd, bs = [list(carry[0]), list(carry[1])] + list(carry[2:])
        s16 = m * 16
        for i in range(16):
            a = [jnp.zeros((_TPW,), jnp.float32) for _ in range(3)]
            p = [jnp.zeros((_TPW,), jnp.float32) for _ in range(3)]
            mm = [jnp.zeros((_TPW,), jnp.float32) for _ in range(3)]
            for j in range(_D):
                t = xT_v[j, :] + qs[j]
                r = _round_ne(t)
                e = t - r
                u = j % 3
                a[u] = a[u] + e * e
                p[u] = p[u] + r
                mm[u] = jnp.maximum(mm[u], jnp.abs(e))
            acc = (a[0] + a[1]) + a[2]
            ps = (p[0] + p[1]) + p[2]
            em = jnp.maximum(jnp.maximum(mm[0], mm[1]), mm[2])
            h = ps * 0.5
            odd = jnp.abs(h - _round_ne(h)) * 2.0
            score = acc + (1.0 - 2.0 * em) * odd
            better = score < bd
            bd = jnp.where(better, score, bd)
            bs = jnp.where(better, _lanes(s16 + i), bs)
            if i < 15:
                b = _INNER_TZ[i]
                for j in _SUPPORTS[b]:
                    qs[j] = qs[j] + sb[b]
                sb[b] = -sb[b]
            else:
                d0 = delta_v[pl.ds(m * _D, _NSUB)]
                d1 = delta_v[pl.ds(m * _D + 8, _NSUB)]
                for j in range(_D):
                    dj = _lanes(d0[j] if j < _NSUB else d1[j - 8])
                    qs[j] = qs[j] + dj
        return (tuple(qs), tuple(sb), bd, bs)

    _, _, bd, bs = lax.fori_loop(
        0, _NBLK, mbody, (tuple(qs), tuple(sb), bd, bs))

    g = bs ^ (bs >> 1)
    bk = g & 4096
    for b in range(12):
        bk = bk | (((g >> b) & 1) << (11 - b))

    bd_v[...] = bd
    bk_v[...] = bk
    pltpu.sync_copy(bd_v, bd_hbm.at[pl.ds(base, _TPW)])
    pltpu.sync_copy(bk_v, bk_hbm.at[pl.ds(base, _TPW)])

    descs = []
    for j in range(_D):
        idx = bk * _D + j
        descs.append(pltpu.async_copy(cflat_hbm.at[idx], crowT_v.at[j], sem))
    for d in descs:
        d.wait()

    ps = jnp.zeros((_TPW,), jnp.float32)
    mx = jnp.full((_TPW,), -1.0, jnp.float32)
    col = jnp.zeros((_TPW,), jnp.int32)
    for j in range(_D):
        cj = crowT_v[j, :]
        t = xT_v[j, :] - cj
        r = _round_ne(t)
        ea = jnp.abs(t - r)
        ps = ps + r
        upd = ea > mx
        mx = jnp.where(upd, ea, mx)
        col = jnp.where(upd, _lanes(jnp.int32(j)), col)
    h = ps * 0.5
    odd = jnp.abs(h - _round_ne(h)) * 2.0 > 0.5
    for j in range(_D):
        cj = crowT_v[j, :]
        t = xT_v[j, :] - cj
        r = _round_ne(t)
        e = t - r
        f = jnp.where((col == j) & odd, r + jnp.sign(e), r)
        outT_v[j, :] = (f + cj) * _Y

    pltpu.sync_copy(outT_v, outT_hbm.at[:, pl.ds(base, _TPW)])


@functools.cache
def _make_sc_call(interpret=False):
    mesh = plsc.VectorSubcoreMesh(core_axis_name="c", subcore_axis_name="s",
                                  num_cores=_NCORE, num_subcores=_NSUB)
    return pl.kernel(
        _sc_body,
        out_type=(
            jax.ShapeDtypeStruct((_D, _NTOK), jnp.float32),
            jax.ShapeDtypeStruct((_NTOK,), jnp.float32),
            jax.ShapeDtypeStruct((_NTOK,), jnp.int32),
        ),
        mesh=mesh,
        scratch_types=[
            pltpu.VMEM((_D, _TPW), jnp.float32),
            pltpu.VMEM((_NBLK * _D,), jnp.float32),
            pltpu.VMEM((_D, _TPW), jnp.float32),
            pltpu.VMEM((_D, _TPW), jnp.float32),
            pltpu.VMEM((_TPW,), jnp.float32),
            pltpu.VMEM((_TPW,), jnp.int32),
            pltpu.SemaphoreType.DMA,
        ],
        compiler_params=pltpu.CompilerParams(use_tc_tiling_on_sc=False),
        interpret=interpret,
    )


def _tc_body(x4_ref, c4t_ref, kmap_ref, g12h_ref, u4_ref,
             y_ref, bd_ref, bk_ref, bda_ref, bka_ref):
    def tb_body(tb, _):
        xb = x4_ref[pl.ds(tb * 8, 8), :]
        xjs = [jnp.broadcast_to(xb[:, j:j + 1], (8, _KB))
               for j in range(_D)]

        def kb_body(kb, carry):
            bd, bk = carry
            a = jnp.zeros((8, _KB), jnp.float32)
            p = jnp.zeros((8, _KB), jnp.float32)
            mm = jnp.zeros((8, _KB), jnp.float32)
            for j in range(_D):
                cj = c4t_ref[pl.ds(j, 1), pl.ds(kb * _KB, _KB)]
                t = xjs[j] - cj
                r = _round_ne(t)
                e = t - r
                a = a + e * e
                p = p + r
                mm = jnp.maximum(mm, jnp.abs(e))
            h = p * 0.5
            odd = jnp.abs(h - _round_ne(h)) * 2.0
            score = a + (1.0 - 2.0 * mm) * odd
            kv = jnp.broadcast_to(
                kmap_ref[pl.ds(0, 1), pl.ds(kb * _KB, _KB)], (8, _KB))
            better = score < bd
            return (jnp.where(better, score, bd),
                    jnp.where(better, kv, bk))

        bd, bk = lax.fori_loop(
            0, _NKB, kb_body,
            (jnp.full((8, _KB), jnp.inf, jnp.float32),
             jnp.zeros((8, _KB), jnp.int32)))
        bda_ref[pl.ds(tb * 8, 8), :] = bd
        bka_ref[pl.ds(tb * 8, 8), :] = bk
        return 0

    lax.fori_loop(0, _NTOK // 8, tb_body, 0)

    bd = bda_ref[...]
    bk = bka_ref[...]
    m = jnp.min(bd, axis=1, keepdims=True)
    kw = jnp.min(jnp.where(bd == m, bk, jnp.int32(1 << 30)),
                 axis=1, keepdims=True)

    c4w = jnp.where(kw >= 4096, u4_ref[pl.ds(0, 1), :], 0.0)
    for b in range(12):
        bit = ((kw >> (11 - b)) & 1).astype(jnp.float32)
        c4w = c4w + bit * g12h_ref[pl.ds(b, 1), :]
    i24 = lax.broadcasted_iota(jnp.int32, (_NTOK, _D), 1)
    xall = x4_ref[...]
    t = xall - c4w
    r = _round_ne(t)
    e = t - r
    ea = jnp.abs(e)
    psum = jnp.sum(r, axis=1, keepdims=True)
    hh = psum * 0.5
    odd = jnp.abs(hh - _round_ne(hh)) * 2.0 > 0.5
    mx = jnp.max(ea, axis=1, keepdims=True)
    colv = jnp.min(jnp.where(ea == mx, i24, 999), axis=1, keepdims=True)
    f = jnp.where((i24 == colv) & odd, r + jnp.sign(e), r)
    y_ref[...] = (f + c4w) * _Y
    bd_ref[...] = m
    bk_ref[...] = kw


@functools.cache
def _make_tc_call():
    return pl.pallas_call(
        _tc_body,
        out_shape=(
            jax.ShapeDtypeStruct((_NTOK, _D), jnp.float32),
            jax.ShapeDtypeStruct((_NTOK, 1), jnp.float32),
            jax.ShapeDtypeStruct((_NTOK, 1), jnp.int32),
        ),
        scratch_shapes=[
            pltpu.VMEM((_NTOK, _KB), jnp.float32),
            pltpu.VMEM((_NTOK, _KB), jnp.int32),
        ],
    )


@jax.jit
def kernel(x_in, C_rep):
    x = x_in / _A
    x4 = x * 0.25
    xT4 = x4.T
    c4 = C_rep.astype(jnp.float32) * 0.25
    cflat = c4.reshape(-1)
    delta = jnp.concatenate(
        [c4[_KEND] - c4[_KNXT], jnp.zeros((1, _D), jnp.float32)], axis=0)
    c4t_tc = c4.T[:, _PERM[_SSC:]]
    kmap = jnp.asarray(_PERM[_SSC:][None, :])

    yT_sc, bd_sc, bk_sc = _make_sc_call()(xT4, cflat, delta.reshape(-1))
    y_tc, bd_tc, bk_tc = _make_tc_call()(
        x4, c4t_tc, kmap, jnp.asarray(_G12H), jnp.asarray(_U4)[None, :])

    bd_tc = bd_tc[:, 0]
    bk_tc = bk_tc[:, 0]
    use_tc = (bd_tc < bd_sc) | ((bd_tc == bd_sc) & (bk_tc < bk_sc))
    return jnp.where(use_tc[:, None], y_tc, yT_sc.T)


# speedup vs baseline: 1.2880x; 1.2868x over previous
import functools
import numpy as np
import jax
import jax.numpy as jnp
from jax import lax
from jax.experimental import pallas as pl
from jax.experimental.pallas import tpu as pltpu
from jax.experimental.pallas import tpu_sc as plsc

_A = np.float32(1.0 / np.sqrt(8.0))
_MAGIC = np.float32(1.5 * 2.0**23)
_Y = np.float32(4.0) * _A
_NTOK = 512
_K = 8192
_D = 24
_NCORE = 2
_NSUB = 16
_NW = _NCORE * _NSUB
_TPW = _NTOK // _NW

_SSC = 2560
_NBLK = _SSC // 16
_NTC = _K - _SSC
_KB = 512
_NKB = _NTC // _KB

_SUPPORTS = (
    (0, 1, 2, 3, 4, 5, 6, 7),
    (0, 1, 2, 3, 8, 9, 10, 11),
    (0, 1, 4, 5, 8, 9, 12, 13),
    (0, 2, 4, 6, 8, 10, 12, 14),
)
_INNER_TZ = tuple((i + 1 & -(i + 1)).bit_length() - 1 for i in range(15))

_G12 = np.array([
    [1,1,1,1,1,1,1,1,0,0,0,0,0,0,0,0,0,0,0,0,0,0,0,0],
    [1,1,1,1,0,0,0,0,1,1,1,1,0,0,0,0,0,0,0,0,0,0,0,0],
    [1,1,0,0,1,1,0,0,1,1,0,0,1,1,0,0,0,0,0,0,0,0,0,0],
    [1,0,1,0,1,0,1,0,1,0,1,0,1,0,1,0,0,0,0,0,0,0,0,0],
    [1,0,0,1,1,0,0,1,1,0,0,1,1,0,0,1,0,0,0,0,0,0,0,0],
    [1,0,1,0,1,0,0,1,1,1,0,0,0,0,0,0,1,1,0,0,0,0,0,0],
    [1,0,0,1,1,1,0,0,1,0,1,0,0,0,0,0,1,0,1,0,0,0,0,0],
    [1,1,0,0,1,0,1,0,1,0,0,1,0,0,0,0,1,0,0,1,0,0,0,0],
    [0,1,1,1,1,0,0,0,1,0,0,0,1,0,0,0,1,0,0,0,1,0,0,0],
    [0,0,0,0,0,0,0,0,1,1,0,0,1,1,0,0,1,1,0,0,1,1,0,0],
    [0,0,0,0,0,0,0,0,1,0,1,0,1,0,1,0,1,0,1,0,1,0,1,0],
    [1,1,1,1,1,1,1,1,1,1,1,1,1,1,1,1,1,1,1,1,1,1,1,1]], dtype=np.int64)
_G12H = (_G12 * 0.5).astype(np.float32)
_U4 = (np.array([-3] + [1] * 23) * 0.25).astype(np.float32)


def _gray_to_k(g):
    k = g & 4096
    for b in range(12):
        k = k | (((g >> b) & 1) << (11 - b))
    return k


def _np_step_to_k(s):
    g = s ^ (s >> 1)
    return _gray_to_k(g)


_PERM = np.array([_np_step_to_k(s) for s in range(_K)], dtype=np.int32)
_KEND = _PERM[np.arange(_NBLK - 1) * 16 + 15]
_KNXT = _PERM[np.arange(_NBLK - 1) * 16 + 16]


def _round_ne(t):
    return (t + _MAGIC) - _MAGIC


def _lanes(x):
    return jnp.broadcast_to(x, (_NSUB,))


def _sc_body(xT_hbm, cflat_hbm, delta_hbm, outT_hbm, bd_hbm, bk_hbm,
             xT_v, delta_v, outT_v, crowT_v, bd_v, bk_v, sem):
    cid = lax.axis_index("c")
    sid = lax.axis_index("s")
    wid = sid * _NCORE + cid
    base = wid * _TPW

    pltpu.sync_copy(xT_hbm.at[:, pl.ds(base, _TPW)], xT_v)
    pltpu.sync_copy(delta_hbm, delta_v)

    qs = [jnp.zeros((_TPW,), jnp.float32) for _ in range(_D)]
    sb = [jnp.full((_TPW,), -0.5, jnp.float32) for _ in range(4)]
    bd = jnp.full((_TPW,), jnp.inf, jnp.float32)
    bs = jnp.zeros((_TPW,), jnp.int32)

    def mbody(m, carry):
        qs, sb, bd, bs = [list(carry[0]), list(carry[1])] + list(carry[2:])
        s16 = m * 16
        for i in range(16):
            a = [jnp.zeros((_TPW,), jnp.float32) for _ in range(3)]
            p = [jnp.zeros((_TPW,), jnp.float32) for _ in range(3)]
            mm = [jnp.zeros((_TPW,), jnp.float32) for _ in range(3)]
            for j in range(_D):
                t = xT_v[j, :] + qs[j]
                r = _round_ne(t)
                e = t - r
                u = j % 3
                a[u] = a[u] + e * e
                p[u] = p[u] + r
                mm[u] = jnp.maximum(mm[u], jnp.abs(e))
            acc = (a[0] + a[1]) + a[2]
            ps = (p[0] + p[1]) + p[2]
            em = jnp.maximum(jnp.maximum(mm[0], mm[1]), mm[2])
            h = ps * 0.5
            odd = jnp.abs(h - _round_ne(h)) * 2.0
            score = acc + (1.0 - 2.0 * em) * odd
            better = score < bd
            bd = jnp.where(better, score, bd)
            bs = jnp.where(better, _lanes(s16 + i), bs)
            if i < 15:
                b = _INNER_TZ[i]
                for j in _SUPPORTS[b]:
                    qs[j] = qs[j] + sb[b]
                sb[b] = -sb[b]
            else:
                d0 = delta_v[pl.ds(m * _D, _NSUB)]
                d1 = delta_v[pl.ds(m * _D + 8, _NSUB)]
                for j in range(_D):
                    dj = _lanes(d0[j] if j < _NSUB else d1[j - 8])
                    qs[j] = qs[j] + dj
        return (tuple(qs), tuple(sb), bd, bs)

    _, _, bd, bs = lax.fori_loop(
        0, _NBLK, mbody, (tuple(qs), tuple(sb), bd, bs))

    g = bs ^ (bs >> 1)
    bk = g & 4096
    for b in range(12):
        bk = bk | (((g >> b) & 1) << (11 - b))

    bd_v[...] = bd
    bk_v[...] = bk
    pltpu.sync_copy(bd_v, bd_hbm.at[pl.ds(base, _TPW)])
    pltpu.sync_copy(bk_v, bk_hbm.at[pl.ds(base, _TPW)])

    descs = []
    for j in range(_D):
        idx = bk * _D + j
        descs.append(pltpu.async_copy(cflat_hbm.at[idx], crowT_v.at[j], sem))
    for d in descs:
        d.wait()

    ps = jnp.zeros((_TPW,), jnp.float32)
    mx = jnp.full((_TPW,), -1.0, jnp.float32)
    col = jnp.zeros((_TPW,), jnp.int32)
    for j in range(_D):
        cj = crowT_v[j, :]
        t = xT_v[j, :] - cj
        r = _round_ne(t)
        ea = jnp.abs(t - r)
        ps = ps + r
        upd = ea > mx
        mx = jnp.where(upd, ea, mx)
        col = jnp.where(upd, _lanes(jnp.int32(j)), col)
    h = ps * 0.5
    odd = jnp.abs(h - _round_ne(h)) * 2.0 > 0.5
    for j in range(_D):
        cj = crowT_v[j, :]
        t = xT_v[j, :] - cj
        r = _round_ne(t)
        e = t - r
        f = jnp.where((col == j) & odd, r + jnp.sign(e), r)
        outT_v[j, :] = (f + cj) * _Y

    pltpu.sync_copy(outT_v, outT_hbm.at[:, pl.ds(base, _TPW)])


@functools.cache
def _make_sc_call(interpret=False):
    mesh = plsc.VectorSubcoreMesh(core_axis_name="c", subcore_axis_name="s",
                                  num_cores=_NCORE, num_subcores=_NSUB)
    return pl.kernel(
        _sc_body,
        out_type=(
            jax.ShapeDtypeStruct((_D, _NTOK), jnp.float32),
            jax.ShapeDtypeStruct((_NTOK,), jnp.float32),
            jax.ShapeDtypeStruct((_NTOK,), jnp.int32),
        ),
        mesh=mesh,
        scratch_types=[
            pltpu.VMEM((_D, _TPW), jnp.float32),
            pltpu.VMEM((_NBLK * _D,), jnp.float32),
            pltpu.VMEM((_D, _TPW), jnp.float32),
            pltpu.VMEM((_D, _TPW), jnp.float32),
            pltpu.VMEM((_TPW,), jnp.float32),
            pltpu.VMEM((_TPW,), jnp.int32),
            pltpu.SemaphoreType.DMA,
        ],
        compiler_params=pltpu.CompilerParams(use_tc_tiling_on_sc=False),
        interpret=interpret,
    )


def _tc_body(x4_ref, c4t_ref, kmap_ref, g12h_ref, u4_ref,
             y_ref, bd_ref, bk_ref, bda_ref, bka_ref):
    def tb_body(tb, _):
        xb = x4_ref[pl.ds(tb * 8, 8), :]
        xjs = [jnp.broadcast_to(xb[:, j:j + 1], (8, _KB))
               for j in range(_D)]

        def kb_body(kb, carry):
            bd, bk = carry
            a = jnp.zeros((8, _KB), jnp.float32)
            p = jnp.zeros((8, _KB), jnp.float32)
            mm = jnp.zeros((8, _KB), jnp.float32)
            for j in range(_D):
                cj = c4t_ref[pl.ds(j, 1), pl.ds(kb * _KB, _KB)]
                t = xjs[j] - cj
                r = _round_ne(t)
                e = t - r
                a = a + e * e
                p = p + r
                mm = jnp.maximum(mm, jnp.abs(e))
            h = p * 0.5
            odd = jnp.abs(h - _round_ne(h)) * 2.0
            score = a + (1.0 - 2.0 * mm) * odd
            kv = jnp.broadcast_to(
                kmap_ref[pl.ds(0, 1), pl.ds(kb * _KB, _KB)], (8, _KB))
            better = score < bd
            return (jnp.where(better, score, bd),
                    jnp.where(better, kv, bk))

        bd, bk = lax.fori_loop(
            0, _NKB, kb_body,
            (jnp.full((8, _KB), jnp.inf, jnp.float32),
             jnp.zeros((8, _KB), jnp.int32)))
        bda_ref[pl.ds(tb * 8, 8), :] = bd
        bka_ref[pl.ds(tb * 8, 8), :] = bk
        return 0

    lax.fori_loop(0, _NTOK // 8, tb_body, 0)

    bd = bda_ref[...]
    bk = bka_ref[...]
    m = jnp.min(bd, axis=1, keepdims=True)
    kw = jnp.min(jnp.where(bd == m, bk, jnp.int32(1 << 30)),
                 axis=1, keepdims=True)

    c4w = jnp.where(kw >= 4096, u4_ref[pl.ds(0, 1), :], 0.0)
    for b in range(12):
        bit = ((kw >> (11 - b)) & 1).astype(jnp.float32)
        c4w = c4w + bit * g12h_ref[pl.ds(b, 1), :]
    i24 = lax.broadcasted_iota(jnp.int32, (_NTOK, _D), 1)
    xall = x4_ref[...]
    t = xall - c4w
    r = _round_ne(t)
    e = t - r
    ea = jnp.abs(e)
    psum = jnp.sum(r, axis=1, keepdims=True)
    hh = psum * 0.5
    odd = jnp.abs(hh - _round_ne(hh)) * 2.0 > 0.5
    mx = jnp.max(ea, axis=1, keepdims=True)
    colv = jnp.min(jnp.where(ea == mx, i24, 999), axis=1, keepdims=True)
    f = jnp.where((i24 == colv) & odd, r + jnp.sign(e), r)
    y_ref[...] = (f + c4w) * _Y
    bd_ref[...] = m
    bk_ref[...] = kw


@functools.cache
def _make_tc_call():
    return pl.pallas_call(
        _tc_body,
        out_shape=(
            jax.ShapeDtypeStruct((_NTOK, _D), jnp.float32),
            jax.ShapeDtypeStruct((_NTOK, 1), jnp.float32),
            jax.ShapeDtypeStruct((_NTOK, 1), jnp.int32),
        ),
        scratch_shapes=[
            pltpu.VMEM((_NTOK, _KB), jnp.float32),
            pltpu.VMEM((_NTOK, _KB), jnp.int32),
        ],
    )


@jax.jit
def kernel(x_in, C_rep):
    x = x_in / _A
    x4 = x * 0.25
    xT4 = x4.T
    c4 = C_rep.astype(jnp.float32) * 0.25
    cflat = c4.reshape(-1)
    delta = jnp.concatenate(
        [c4[_KEND] - c4[_KNXT], jnp.zeros((1, _D), jnp.float32)], axis=0)
    c4t_tc = c4.T[:, _PERM[_SSC:]]
    kmap = jnp.asarray(_PERM[_SSC:][None, :])

    yT_sc, bd_sc, bk_sc = _make_sc_call()(xT4, cflat, delta.reshape(-1))
    y_tc, bd_tc, bk_tc = _make_tc_call()(
        x4, c4t_tc, kmap, jnp.asarray(_G12H), jnp.asarray(_U4)[None, :])

    bd_tc = bd_tc[:, 0]
    bk_tc = bk_tc[:, 0]
    use_tc = (bd_tc < bd_sc) | ((bd_tc == bd_sc) & (bk_tc < bk_sc))
    return jnp.where(use_tc[:, None], y_tc, yT_sc.T)


# speedup vs baseline: 1.3571x; 1.0536x over previous
import functools
import numpy as np
import jax
import jax.numpy as jnp
from jax import lax
from jax.experimental import pallas as pl
from jax.experimental.pallas import tpu as pltpu
from jax.experimental.pallas import tpu_sc as plsc

_A = np.float32(1.0 / np.sqrt(8.0))
_MAGIC = np.float32(1.5 * 2.0**23)
_Y = np.float32(4.0) * _A
_NTOK = 512
_K = 8192
_D = 24
_NCORE = 2
_NSUB = 16
_NW = _NCORE * _NSUB
_TPW = _NTOK // _NW

_SSC = 2048
_NBLK = _SSC // 16
_NTC = _K - _SSC
_KB = 512
_NKB = _NTC // _KB

_SUPPORTS = (
    (0, 1, 2, 3, 4, 5, 6, 7),
    (0, 1, 2, 3, 8, 9, 10, 11),
    (0, 1, 4, 5, 8, 9, 12, 13),
    (0, 2, 4, 6, 8, 10, 12, 14),
)
_INNER_TZ = tuple((i + 1 & -(i + 1)).bit_length() - 1 for i in range(15))

_G12 = np.array([
    [1,1,1,1,1,1,1,1,0,0,0,0,0,0,0,0,0,0,0,0,0,0,0,0],
    [1,1,1,1,0,0,0,0,1,1,1,1,0,0,0,0,0,0,0,0,0,0,0,0],
    [1,1,0,0,1,1,0,0,1,1,0,0,1,1,0,0,0,0,0,0,0,0,0,0],
    [1,0,1,0,1,0,1,0,1,0,1,0,1,0,1,0,0,0,0,0,0,0,0,0],
    [1,0,0,1,1,0,0,1,1,0,0,1,1,0,0,1,0,0,0,0,0,0,0,0],
    [1,0,1,0,1,0,0,1,1,1,0,0,0,0,0,0,1,1,0,0,0,0,0,0],
    [1,0,0,1,1,1,0,0,1,0,1,0,0,0,0,0,1,0,1,0,0,0,0,0],
    [1,1,0,0,1,0,1,0,1,0,0,1,0,0,0,0,1,0,0,1,0,0,0,0],
    [0,1,1,1,1,0,0,0,1,0,0,0,1,0,0,0,1,0,0,0,1,0,0,0],
    [0,0,0,0,0,0,0,0,1,1,0,0,1,1,0,0,1,1,0,0,1,1,0,0],
    [0,0,0,0,0,0,0,0,1,0,1,0,1,0,1,0,1,0,1,0,1,0,1,0],
    [1,1,1,1,1,1,1,1,1,1,1,1,1,1,1,1,1,1,1,1,1,1,1,1]], dtype=np.int64)
_G12H = (_G12 * 0.5).astype(np.float32)
_U4 = (np.array([-3] + [1] * 23) * 0.25).astype(np.float32)


def _gray_to_k(g):
    k = g & 4096
    for b in range(12):
        k = k | (((g >> b) & 1) << (11 - b))
    return k


def _np_step_to_k(s):
    g = s ^ (s >> 1)
    return _gray_to_k(g)


_PERM = np.array([_np_step_to_k(s) for s in range(_K)], dtype=np.int32)
_KEND = _PERM[np.arange(_NBLK - 1) * 16 + 15]
_KNXT = _PERM[np.arange(_NBLK - 1) * 16 + 16]


def _round_ne(t):
    return (t + _MAGIC) - _MAGIC


def _lanes(x):
    return jnp.broadcast_to(x, (_NSUB,))


def _sc_body(xT_hbm, cflat_hbm, delta_hbm, outT_hbm, bd_hbm, bk_hbm,
             xT_v, delta_v, outT_v, crowT_v, bd_v, bk_v, sem):
    cid = lax.axis_index("c")
    sid = lax.axis_index("s")
    wid = sid * _NCORE + cid
    base = wid * _TPW

    pltpu.sync_copy(xT_hbm.at[:, pl.ds(base, _TPW)], xT_v)
    pltpu.sync_copy(delta_hbm, delta_v)

    qs = [jnp.zeros((_TPW,), jnp.float32) for _ in range(_D)]
    sb = [jnp.full((_TPW,), -0.5, jnp.float32) for _ in range(4)]
    bd = jnp.full((_TPW,), jnp.inf, jnp.float32)
    bs = jnp.zeros((_TPW,), jnp.int32)

    def mbody(m, carry):
        qs, sb, bd, bs = [list(carry[0]), list(carry[1])] + list(carry[2:])
        s16 = m * 16
        for i in range(16):
            a = [jnp.zeros((_TPW,), jnp.float32) for _ in range(3)]
            p = [jnp.zeros((_TPW,), jnp.float32) for _ in range(3)]
            mm = [jnp.zeros((_TPW,), jnp.float32) for _ in range(3)]
            for j in range(_D):
                t = xT_v[j, :] + qs[j]
                r = _round_ne(t)
                e = t - r
                u = j % 3
                a[u] = a[u] + e * e
                p[u] = p[u] + r
                mm[u] = jnp.maximum(mm[u], jnp.abs(e))
            acc = (a[0] + a[1]) + a[2]
            ps = (p[0] + p[1]) + p[2]
            em = jnp.maximum(jnp.maximum(mm[0], mm[1]), mm[2])
            h = ps * 0.5
            odd = jnp.abs(h - _round_ne(h)) * 2.0
            score = acc + (1.0 - 2.0 * em) * odd
            better = score < bd
            bd = jnp.where(better, score, bd)
            bs = jnp.where(better, _lanes(s16 + i), bs)
            if i < 15:
                b = _INNER_TZ[i]
                for j in _SUPPORTS[b]:
                    qs[j] = qs[j] + sb[b]
                sb[b] = -sb[b]
            else:
                d0 = delta_v[pl.ds(m * _D, _NSUB)]
                d1 = delta_v[pl.ds(m * _D + 8, _NSUB)]
                for j in range(_D):
                    dj = _lanes(d0[j] if j < _NSUB else d1[j - 8])
                    qs[j] = qs[j] + dj
        return (tuple(qs), tuple(sb), bd, bs)

    _, _, bd, bs = lax.fori_loop(
        0, _NBLK, mbody, (tuple(qs), tuple(sb), bd, bs))

    g = bs ^ (bs >> 1)
    bk = g & 4096
    for b in range(12):
        bk = bk | (((g >> b) & 1) << (11 - b))

    bd_v[...] = bd
    bk_v[...] = bk
    pltpu.sync_copy(bd_v, bd_hbm.at[pl.ds(base, _TPW)])
    pltpu.sync_copy(bk_v, bk_hbm.at[pl.ds(base, _TPW)])

    descs = []
    for j in range(_D):
        idx = bk * _D + j
        descs.append(pltpu.async_copy(cflat_hbm.at[idx], crowT_v.at[j], sem))
    for d in descs:
        d.wait()

    ps = jnp.zeros((_TPW,), jnp.float32)
    mx = jnp.full((_TPW,), -1.0, jnp.float32)
    col = jnp.zeros((_TPW,), jnp.int32)
    for j in range(_D):
        cj = crowT_v[j, :]
        t = xT_v[j, :] - cj
        r = _round_ne(t)
        ea = jnp.abs(t - r)
        ps = ps + r
        upd = ea > mx
        mx = jnp.where(upd, ea, mx)
        col = jnp.where(upd, _lanes(jnp.int32(j)), col)
    h = ps * 0.5
    odd = jnp.abs(h - _round_ne(h)) * 2.0 > 0.5
    for j in range(_D):
        cj = crowT_v[j, :]
        t = xT_v[j, :] - cj
        r = _round_ne(t)
        e = t - r
        f = jnp.where((col == j) & odd, r + jnp.sign(e), r)
        outT_v[j, :] = (f + cj) * _Y

    pltpu.sync_copy(outT_v, outT_hbm.at[:, pl.ds(base, _TPW)])


@functools.cache
def _make_sc_call(interpret=False):
    mesh = plsc.VectorSubcoreMesh(core_axis_name="c", subcore_axis_name="s",
                                  num_cores=_NCORE, num_subcores=_NSUB)
    return pl.kernel(
        _sc_body,
        out_type=(
            jax.ShapeDtypeStruct((_D, _NTOK), jnp.float32),
            jax.ShapeDtypeStruct((_NTOK,), jnp.float32),
            jax.ShapeDtypeStruct((_NTOK,), jnp.int32),
        ),
        mesh=mesh,
        scratch_types=[
            pltpu.VMEM((_D, _TPW), jnp.float32),
            pltpu.VMEM((_NBLK * _D,), jnp.float32),
            pltpu.VMEM((_D, _TPW), jnp.float32),
            pltpu.VMEM((_D, _TPW), jnp.float32),
            pltpu.VMEM((_TPW,), jnp.float32),
            pltpu.VMEM((_TPW,), jnp.int32),
            pltpu.SemaphoreType.DMA,
        ],
        compiler_params=pltpu.CompilerParams(use_tc_tiling_on_sc=False),
        interpret=interpret,
    )


def _tc_body(x4_ref, c4t_ref, kmap_ref, g12h_ref, u4_ref,
             y_ref, bd_ref, bk_ref, bda_ref, bka_ref):
    def tb_body(tb, _):
        xb = x4_ref[pl.ds(tb * 8, 8), :]
        xjs = [jnp.broadcast_to(xb[:, j:j + 1], (8, _KB))
               for j in range(_D)]

        def kb_body(kb, carry):
            bd, bk = carry
            a = jnp.zeros((8, _KB), jnp.float32)
            p = jnp.zeros((8, _KB), jnp.float32)
            mm = jnp.zeros((8, _KB), jnp.float32)
            for j in range(_D):
                cj = c4t_ref[pl.ds(j, 1), pl.ds(kb * _KB, _KB)]
                t = xjs[j] - cj
                r = _round_ne(t)
                e = t - r
                a = a + e * e
                p = p + r
                mm = jnp.maximum(mm, jnp.abs(e))
            h = p * 0.5
            odd = jnp.abs(h - _round_ne(h)) * 2.0
            score = a + (1.0 - 2.0 * mm) * odd
            kv = jnp.broadcast_to(
                kmap_ref[pl.ds(0, 1), pl.ds(kb * _KB, _KB)], (8, _KB))
            better = score < bd
            return (jnp.where(better, score, bd),
                    jnp.where(better, kv, bk))

        bd, bk = lax.fori_loop(
            0, _NKB, kb_body,
            (jnp.full((8, _KB), jnp.inf, jnp.float32),
             jnp.zeros((8, _KB), jnp.int32)))
        bda_ref[pl.ds(tb * 8, 8), :] = bd
        bka_ref[pl.ds(tb * 8, 8), :] = bk
        return 0

    lax.fori_loop(0, _NTOK // 8, tb_body, 0)

    bd = bda_ref[...]
    bk = bka_ref[...]
    m = jnp.min(bd, axis=1, keepdims=True)
    kw = jnp.min(jnp.where(bd == m, bk, jnp.int32(1 << 30)),
                 axis=1, keepdims=True)

    c4w = jnp.where(kw >= 4096, u4_ref[pl.ds(0, 1), :], 0.0)
    for b in range(12):
        bit = ((kw >> (11 - b)) & 1).astype(jnp.float32)
        c4w = c4w + bit * g12h_ref[pl.ds(b, 1), :]
    i24 = lax.broadcasted_iota(jnp.int32, (_NTOK, _D), 1)
    xall = x4_ref[...]
    t = xall - c4w
    r = _round_ne(t)
    e = t - r
    ea = jnp.abs(e)
    psum = jnp.sum(r, axis=1, keepdims=True)
    hh = psum * 0.5
    odd = jnp.abs(hh - _round_ne(hh)) * 2.0 > 0.5
    mx = jnp.max(ea, axis=1, keepdims=True)
    colv = jnp.min(jnp.where(ea == mx, i24, 999), axis=1, keepdims=True)
    f = jnp.where((i24 == colv) & odd, r + jnp.sign(e), r)
    y_ref[...] = (f + c4w) * _Y
    bd_ref[...] = m
    bk_ref[...] = kw


@functools.cache
def _make_tc_call():
    return pl.pallas_call(
        _tc_body,
        out_shape=(
            jax.ShapeDtypeStruct((_NTOK, _D), jnp.float32),
            jax.ShapeDtypeStruct((_NTOK, 1), jnp.float32),
            jax.ShapeDtypeStruct((_NTOK, 1), jnp.int32),
        ),
        scratch_shapes=[
            pltpu.VMEM((_NTOK, _KB), jnp.float32),
            pltpu.VMEM((_NTOK, _KB), jnp.int32),
        ],
    )


@jax.jit
def kernel(x_in, C_rep):
    x = x_in / _A
    x4 = x * 0.25
    xT4 = x4.T
    c4 = C_rep.astype(jnp.float32) * 0.25
    cflat = c4.reshape(-1)
    delta = jnp.concatenate(
        [c4[_KEND] - c4[_KNXT], jnp.zeros((1, _D), jnp.float32)], axis=0)
    c4t_tc = c4.T[:, _PERM[_SSC:]]
    kmap = jnp.asarray(_PERM[_SSC:][None, :])

    yT_sc, bd_sc, bk_sc = _make_sc_call()(xT4, cflat, delta.reshape(-1))
    y_tc, bd_tc, bk_tc = _make_tc_call()(
        x4, c4t_tc, kmap, jnp.asarray(_G12H), jnp.asarray(_U4)[None, :])

    bd_tc = bd_tc[:, 0]
    bk_tc = bk_tc[:, 0]
    use_tc = (bd_tc < bd_sc) | ((bd_tc == bd_sc) & (bk_tc < bk_sc))
    return jnp.where(use_tc[:, None], y_tc, yT_sc.T)
